# baseline (speedup 1.0000x reference)
import functools

import jax
import jax.numpy as jnp
from jax import lax
from jax.experimental import pallas as pl
from jax.experimental.pallas import tpu as pltpu
from jax.experimental.pallas import tpu_sc as plsc

N_SIDE = 1000000
S_EMB = 32
D_SIDE = 64
AA_H = 152
B = 4096

_PACK = 128 // S_EMB
_ROWS = N_SIDE // _PACK
_REP = (AA_H * D_SIDE) // 128


@functools.cache
def _make_sc_gather():
    info = plsc.get_sparse_core_info()
    nw = info.num_cores * info.num_subcores
    b_per_w = B // nw
    mesh = plsc.VectorSubcoreMesh(core_axis_name="c", subcore_axis_name="s")

    @functools.partial(
        pl.kernel,
        mesh=mesh,
        out_type=jax.ShapeDtypeStruct((B, 128), jnp.float32),
        scratch_types=[
            pltpu.VMEM((b_per_w,), jnp.int32),
            pltpu.VMEM((b_per_w,), jnp.int32),
            pltpu.VMEM((b_per_w, 128), jnp.float32),
            pltpu.SemaphoreType.DMA,
        ],
    )
    def gather_k(idx_hbm, table_hbm, out_hbm, idx_v, idx_hi_v, rows_v, sem):
        wid = lax.axis_index("s") * info.num_cores + lax.axis_index("c")
        base = wid * b_per_w
        pltpu.sync_copy(idx_hbm.at[pl.ds(base, b_per_w)], idx_v)
        for j in range(b_per_w // 16):
            sl = pl.ds(j * 16, 16)
            idx_hi_v[sl] = idx_v[sl] >> 2
        pltpu.async_copy(table_hbm.at[idx_hi_v], rows_v, sem).wait()
        pltpu.sync_copy(rows_v, out_hbm.at[pl.ds(base, b_per_w)])

    return gather_k


def _mlp_tile_body(emb_ref, sel_ref, w1s_ref, b1_ref, gamma_ref, beta_ref,
                   w2t_ref, b2_ref, out_ref):
    emb = emb_ref[...]
    sel = sel_ref[...] & (_PACK - 1)
    lane_grp = lax.broadcasted_iota(jnp.int32, emb.shape, 1) >> 5
    mask = (lane_grp == sel).astype(jnp.float32)
    h = jnp.dot(emb * mask, w1s_ref[...], preferred_element_type=jnp.float32)
    h = h + b1_ref[...]
    mu = jnp.mean(h, axis=1, keepdims=True)
    var = jnp.mean((h - mu) ** 2, axis=1, keepdims=True)
    h = (h - mu) * lax.rsqrt(var + 1e-5) * gamma_ref[...] + beta_ref[...]
    h = jnp.maximum(h, 0.0)
    h = jnp.dot(h, w2t_ref[...], preferred_element_type=jnp.float32)
    h = h + b2_ref[...]
    out_ref[...] = jnp.broadcast_to(h[:, None, :], out_ref.shape)


def _tc_mlp_tile(emb, idx2d, w1s, b1, gamma, beta, w2t, b2, bm=256):
    grid = B // bm
    return pl.pallas_call(
        _mlp_tile_body,
        grid=(grid,),
        in_specs=[
            pl.BlockSpec((bm, 128), lambda i: (i, 0)),
            pl.BlockSpec((bm, 1), lambda i: (i, 0)),
            pl.BlockSpec((128, D_SIDE), lambda i: (0, 0)),
            pl.BlockSpec((1, D_SIDE), lambda i: (0, 0)),
            pl.BlockSpec((1, D_SIDE), lambda i: (0, 0)),
            pl.BlockSpec((1, D_SIDE), lambda i: (0, 0)),
            pl.BlockSpec((D_SIDE, D_SIDE), lambda i: (0, 0)),
            pl.BlockSpec((1, D_SIDE), lambda i: (0, 0)),
        ],
        out_specs=pl.BlockSpec((bm, AA_H, D_SIDE), lambda i: (i, 0, 0)),
        out_shape=jax.ShapeDtypeStruct((B, AA_H, D_SIDE), jnp.float32),
    )(emb, idx2d, w1s, b1, gamma, beta, w2t, b2)


def kernel(side, table, W1, b1, gamma, beta, W2, b2):
    idx = side.astype(jnp.int32)
    table128 = table.reshape(_ROWS, 128)
    emb = _make_sc_gather()(idx, table128)
    return _tc_mlp_tile(
        emb,
        idx.reshape(B, 1),
        jnp.tile(W1.T, (_PACK, 1)),
        b1.reshape(1, D_SIDE),
        gamma.reshape(1, D_SIDE),
        beta.reshape(1, D_SIDE),
        W2.T,
        b2.reshape(1, D_SIDE),
    )


# speedup vs baseline: 6.8881x; 6.8881x over previous
import functools

import jax
import jax.numpy as jnp
from jax import lax
from jax.experimental import pallas as pl
from jax.experimental.pallas import tpu as pltpu
from jax.experimental.pallas import tpu_sc as plsc

N_SIDE = 1000000
S_EMB = 32
D_SIDE = 64
AA_H = 152
B = 4096

_A_BLK = 19


@functools.cache
def _make_sc_gather():
    info = plsc.get_sparse_core_info()
    nw = info.num_cores * info.num_subcores
    b_per_w = B // nw
    nbatch = b_per_w // 16
    mesh = plsc.VectorSubcoreMesh(core_axis_name="c", subcore_axis_name="s")

    @functools.partial(
        pl.kernel,
        mesh=mesh,
        out_type=jax.ShapeDtypeStruct((B, S_EMB), jnp.float32),
        scratch_types=[
            pltpu.VMEM((b_per_w,), jnp.int32),
            pltpu.VMEM((16 * S_EMB, 128), jnp.float32),
            pltpu.VMEM((b_per_w, S_EMB), jnp.float32),
            pltpu.SemaphoreType.DMA,
        ],
    )
    def gather_k(idx_hbm, tableT_hbm, out_hbm, idx_v, buf_v, emb_v, sem):
        wid = lax.axis_index("s") * info.num_cores + lax.axis_index("c")
        base = wid * b_per_w
        pltpu.sync_copy(idx_hbm.at[pl.ds(base, b_per_w)], idx_v)
        iota16 = lax.iota(jnp.int32, 16)

        for g in range(nbatch):
            vec = idx_v[pl.ds(g * 16, 16)]
            copies = []
            for j in range(16):
                start = pl.multiple_of((vec[j] >> 7) * 128, 128)
                copies.append(pltpu.async_copy(
                    tableT_hbm.at[:, pl.ds(start, 128)],
                    buf_v.at[pl.ds(j * S_EMB, S_EMB)], sem))
            for cp in copies:
                cp.wait()
            p_all = vec & 15
            for j in range(16):
                jsplat = jnp.full((16,), j, jnp.int32)
                p_splat = p_all[jsplat]
                cj = pl.multiple_of((((vec[j]) & 127) >> 4) * 16, 16)
                for h in range(S_EMB // 16):
                    acc = jnp.zeros((16,), jnp.float32)
                    for d in range(16):
                        chunk = buf_v[j * S_EMB + h * 16 + d, pl.ds(cj, 16)]
                        rot = chunk[p_splat]
                        acc = jnp.where(iota16 == d, rot, acc)
                    emb_v[g * 16 + j, pl.ds(h * 16, 16)] = acc

        pltpu.sync_copy(emb_v, out_hbm.at[pl.ds(base, b_per_w)])

    return gather_k


def _mlp_tile_body(emb_ref, w1_ref, b1_ref, gamma_ref, beta_ref,
                   w2_ref, b2_ref, out_ref, ht_s):
    i = pl.program_id(0)

    @pl.when(i == 0)
    def _compute():
        embT = emb_ref[...].T
        h = jnp.dot(w1_ref[...], embT,
                    preferred_element_type=jnp.float32)
        h = h + b1_ref[...]
        mu = jnp.mean(h, axis=0, keepdims=True)
        var = jnp.mean((h - mu) ** 2, axis=0, keepdims=True)
        h = (h - mu) * lax.rsqrt(var + 1e-5) * gamma_ref[...] + beta_ref[...]
        h = jnp.maximum(h, 0.0)
        h = jnp.dot(w2_ref[...], h, preferred_element_type=jnp.float32)
        ht_s[...] = h + b2_ref[...]

    out_ref[...] = jnp.broadcast_to(ht_s[...][None], out_ref.shape)


def _tc_mlp_tile(emb, W1, b1c, gammac, betac, W2, b2c):
    grid = AA_H // _A_BLK
    return pl.pallas_call(
        _mlp_tile_body,
        grid=(grid,),
        in_specs=[
            pl.BlockSpec((B, S_EMB), lambda i: (0, 0)),
            pl.BlockSpec((D_SIDE, S_EMB), lambda i: (0, 0)),
            pl.BlockSpec((D_SIDE, 1), lambda i: (0, 0)),
            pl.BlockSpec((D_SIDE, 1), lambda i: (0, 0)),
            pl.BlockSpec((D_SIDE, 1), lambda i: (0, 0)),
            pl.BlockSpec((D_SIDE, D_SIDE), lambda i: (0, 0)),
            pl.BlockSpec((D_SIDE, 1), lambda i: (0, 0)),
        ],
        out_specs=pl.BlockSpec((_A_BLK, D_SIDE, B), lambda i: (i, 0, 0)),
        out_shape=jax.ShapeDtypeStruct((AA_H, D_SIDE, B), jnp.float32),
        scratch_shapes=[pltpu.VMEM((D_SIDE, B), jnp.float32)],
    )(emb, W1, b1c, gammac, betac, W2, b2c)


def kernel(side, table, W1, b1, gamma, beta, W2, b2):
    idx = side.astype(jnp.int32)
    emb = _make_sc_gather()(idx, table.T)
    out = _tc_mlp_tile(
        emb,
        W1,
        b1.reshape(D_SIDE, 1),
        gamma.reshape(D_SIDE, 1),
        beta.reshape(D_SIDE, 1),
        W2,
        b2.reshape(D_SIDE, 1),
    )
    return out.transpose(2, 0, 1)


# speedup vs baseline: 7.1266x; 1.0346x over previous
import functools

import jax
import jax.numpy as jnp
from jax import lax
from jax.experimental import pallas as pl
from jax.experimental.pallas import tpu as pltpu
from jax.experimental.pallas import tpu_sc as plsc

N_SIDE = 1000000
S_EMB = 32
D_SIDE = 64
AA_H = 152
B = 4096

_A_BLK = 19


@functools.cache
def _make_sc_gather():
    info = plsc.get_sparse_core_info()
    nw = info.num_cores * info.num_subcores
    b_per_w = B // nw
    nbatch = b_per_w // 16
    mesh = plsc.VectorSubcoreMesh(core_axis_name="c", subcore_axis_name="s")

    @functools.partial(
        pl.kernel,
        mesh=mesh,
        out_type=jax.ShapeDtypeStruct((B, S_EMB), jnp.float32),
        scratch_types=[
            pltpu.VMEM((b_per_w,), jnp.int32),
            pltpu.VMEM((16 * S_EMB, 128), jnp.float32),
            pltpu.VMEM((b_per_w, S_EMB), jnp.float32),
            pltpu.SemaphoreType.DMA,
            pltpu.SemaphoreType.DMA,
        ],
    )
    def gather_k(idx_hbm, tableT_hbm, out_hbm, idx_v, buf_v, emb_v,
                 sem_a, sem_b):
        wid = lax.axis_index("s") * info.num_cores + lax.axis_index("c")
        base = wid * b_per_w
        pltpu.sync_copy(idx_hbm.at[pl.ds(base, b_per_w)], idx_v)
        iota16 = lax.iota(jnp.int32, 16)
        sems = (sem_a, sem_b)
        nb8 = b_per_w // 8
        vecs = [idx_v[pl.ds(t * 16, 16)] for t in range(b_per_w // 16)]

        def fire(b, grp):
            vec, off = vecs[b // 2], (b % 2) * 8
            copies = []
            for j in range(8):
                start = pl.multiple_of((vec[off + j] >> 7) * 128, 128)
                slot = grp * 8 + j
                copies.append(pltpu.async_copy(
                    tableT_hbm.at[:, pl.ds(start, 128)],
                    buf_v.at[pl.ds(slot * S_EMB, S_EMB)], sems[grp]))
            return copies

        def extract(b, grp):
            vec, off = vecs[b // 2], (b % 2) * 8
            p_all = vec & 15
            for j in range(8):
                jsplat = jnp.full((16,), off + j, jnp.int32)
                p_splat = p_all[jsplat]
                cj = pl.multiple_of(((vec[off + j] & 127) >> 4) * 16, 16)
                row0 = (grp * 8 + j) * S_EMB
                for h in range(S_EMB // 16):
                    acc = jnp.zeros((16,), jnp.float32)
                    for d in range(16):
                        chunk = buf_v[row0 + h * 16 + d, pl.ds(cj, 16)]
                        rot = chunk[p_splat]
                        acc = jnp.where(iota16 == d, rot, acc)
                    emb_v[b * 8 + j, pl.ds(h * 16, 16)] = acc

        pending = fire(0, 0)
        for b in range(nb8):
            grp = b % 2
            nxt = fire(b + 1, 1 - grp) if b + 1 < nb8 else []
            for cp in pending:
                cp.wait()
            extract(b, grp)
            pending = nxt

        pltpu.sync_copy(emb_v, out_hbm.at[pl.ds(base, b_per_w)])

    return gather_k


def _mlp_tile_body(emb_ref, w1_ref, b1_ref, gamma_ref, beta_ref,
                   w2_ref, b2_ref, out_ref, ht_s):
    i = pl.program_id(0)

    @pl.when(i == 0)
    def _compute():
        embT = emb_ref[...].T
        h = jnp.dot(w1_ref[...], embT,
                    preferred_element_type=jnp.float32)
        h = h + b1_ref[...]
        mu = jnp.mean(h, axis=0, keepdims=True)
        var = jnp.mean((h - mu) ** 2, axis=0, keepdims=True)
        h = (h - mu) * lax.rsqrt(var + 1e-5) * gamma_ref[...] + beta_ref[...]
        h = jnp.maximum(h, 0.0)
        h = jnp.dot(w2_ref[...], h, preferred_element_type=jnp.float32)
        ht_s[...] = h + b2_ref[...]

    out_ref[...] = jnp.broadcast_to(ht_s[...][None], out_ref.shape)


def _tc_mlp_tile(emb, W1, b1c, gammac, betac, W2, b2c):
    grid = AA_H // _A_BLK
    return pl.pallas_call(
        _mlp_tile_body,
        grid=(grid,),
        in_specs=[
            pl.BlockSpec((B, S_EMB), lambda i: (0, 0)),
            pl.BlockSpec((D_SIDE, S_EMB), lambda i: (0, 0)),
            pl.BlockSpec((D_SIDE, 1), lambda i: (0, 0)),
            pl.BlockSpec((D_SIDE, 1), lambda i: (0, 0)),
            pl.BlockSpec((D_SIDE, 1), lambda i: (0, 0)),
            pl.BlockSpec((D_SIDE, D_SIDE), lambda i: (0, 0)),
            pl.BlockSpec((D_SIDE, 1), lambda i: (0, 0)),
        ],
        out_specs=pl.BlockSpec((_A_BLK, D_SIDE, B), lambda i: (i, 0, 0)),
        out_shape=jax.ShapeDtypeStruct((AA_H, D_SIDE, B), jnp.float32),
        scratch_shapes=[pltpu.VMEM((D_SIDE, B), jnp.float32)],
    )(emb, W1, b1c, gammac, betac, W2, b2c)


def kernel(side, table, W1, b1, gamma, beta, W2, b2):
    idx = side.astype(jnp.int32)
    emb = _make_sc_gather()(idx, table.T)
    out = _tc_mlp_tile(
        emb,
        W1,
        b1.reshape(D_SIDE, 1),
        gamma.reshape(D_SIDE, 1),
        beta.reshape(D_SIDE, 1),
        W2,
        b2.reshape(D_SIDE, 1),
    )
    return out.transpose(2, 0, 1)


# speedup vs baseline: 7.3146x; 1.0264x over previous
import functools

import jax
import jax.numpy as jnp
from jax import lax
from jax.experimental import pallas as pl
from jax.experimental.pallas import tpu as pltpu
from jax.experimental.pallas import tpu_sc as plsc

N_SIDE = 1000000
S_EMB = 32
D_SIDE = 64
AA_H = 152
B = 4096

_A_BLK = 19
_NSPLIT = 2
_HB = B // _NSPLIT


@functools.cache
def _make_sc_gather(nidx):
    info = plsc.get_sparse_core_info()
    nw = info.num_cores * info.num_subcores
    b_per_w = nidx // nw
    mesh = plsc.VectorSubcoreMesh(core_axis_name="c", subcore_axis_name="s")

    @functools.partial(
        pl.kernel,
        mesh=mesh,
        out_type=jax.ShapeDtypeStruct((nidx, S_EMB), jnp.float32),
        scratch_types=[
            pltpu.VMEM((b_per_w,), jnp.int32),
            pltpu.VMEM((16 * S_EMB, 128), jnp.float32),
            pltpu.VMEM((b_per_w, S_EMB), jnp.float32),
            pltpu.SemaphoreType.DMA,
            pltpu.SemaphoreType.DMA,
        ],
    )
    def gather_k(idx_hbm, tableT_hbm, out_hbm, idx_v, buf_v, emb_v,
                 sem_a, sem_b):
        wid = lax.axis_index("s") * info.num_cores + lax.axis_index("c")
        base = wid * b_per_w
        pltpu.sync_copy(idx_hbm.at[pl.ds(base, b_per_w)], idx_v)
        iota16 = lax.iota(jnp.int32, 16)
        sems = (sem_a, sem_b)
        nb8 = b_per_w // 8
        vecs = [idx_v[pl.ds(t * 16, 16)] for t in range(b_per_w // 16)]

        def fire(b, grp):
            vec, off = vecs[b // 2], (b % 2) * 8
            copies = []
            for j in range(8):
                start = pl.multiple_of((vec[off + j] >> 7) * 128, 128)
                slot = grp * 8 + j
                copies.append(pltpu.async_copy(
                    tableT_hbm.at[:, pl.ds(start, 128)],
                    buf_v.at[pl.ds(slot * S_EMB, S_EMB)], sems[grp]))
            return copies

        def extract(b, grp):
            vec, off = vecs[b // 2], (b % 2) * 8
            p_all = vec & 15
            for j in range(8):
                jsplat = jnp.full((16,), off + j, jnp.int32)
                p_splat = p_all[jsplat]
                cj = pl.multiple_of(((vec[off + j] & 127) >> 4) * 16, 16)
                row0 = (grp * 8 + j) * S_EMB
                for h in range(S_EMB // 16):
                    acc = jnp.zeros((16,), jnp.float32)
                    for d in range(16):
                        chunk = buf_v[row0 + h * 16 + d, pl.ds(cj, 16)]
                        rot = chunk[p_splat]
                        acc = jnp.where(iota16 == d, rot, acc)
                    emb_v[b * 8 + j, pl.ds(h * 16, 16)] = acc

        pending = fire(0, 0)
        for b in range(nb8):
            grp = b % 2
            nxt = fire(b + 1, 1 - grp) if b + 1 < nb8 else []
            for cp in pending:
                cp.wait()
            extract(b, grp)
            pending = nxt

        pltpu.sync_copy(emb_v, out_hbm.at[pl.ds(base, b_per_w)])

    return gather_k


def _mlp_body(emb_ref, w1_ref, b1_ref, gamma_ref, beta_ref,
              w2_ref, b2_ref, out_ref, ht_s):
    i = pl.program_id(0)

    @pl.when(i == 0)
    def _compute():
        embT = emb_ref[...].T
        h = jnp.dot(w1_ref[...], embT,
                    preferred_element_type=jnp.float32)
        h = h + b1_ref[...]
        mu = jnp.mean(h, axis=0, keepdims=True)
        var = jnp.mean((h - mu) ** 2, axis=0, keepdims=True)
        h = (h - mu) * lax.rsqrt(var + 1e-5) * gamma_ref[...] + beta_ref[...]
        h = jnp.maximum(h, 0.0)
        h = jnp.dot(w2_ref[...], h, preferred_element_type=jnp.float32)
        ht_s[...] = h + b2_ref[...]

    out_ref[...] = jnp.broadcast_to(ht_s[...][None], out_ref.shape)


def _mlp_body_alias(emb_ref, w1_ref, b1_ref, gamma_ref, beta_ref,
                    w2_ref, b2_ref, prev_ref, out_ref, ht_s):
    del prev_ref
    _mlp_body(emb_ref, w1_ref, b1_ref, gamma_ref, beta_ref,
              w2_ref, b2_ref, out_ref, ht_s)


def _tc_half(emb_h, W1, b1c, gammac, betac, W2, b2c, half, out_prev=None):
    grid = AA_H // _A_BLK
    w_specs = [
        pl.BlockSpec((D_SIDE, S_EMB), lambda i: (0, 0)),
        pl.BlockSpec((D_SIDE, 1), lambda i: (0, 0)),
        pl.BlockSpec((D_SIDE, 1), lambda i: (0, 0)),
        pl.BlockSpec((D_SIDE, 1), lambda i: (0, 0)),
        pl.BlockSpec((D_SIDE, D_SIDE), lambda i: (0, 0)),
        pl.BlockSpec((D_SIDE, 1), lambda i: (0, 0)),
    ]
    in_specs = [pl.BlockSpec((_HB, S_EMB), lambda i: (0, 0))] + w_specs
    args = [emb_h, W1, b1c, gammac, betac, W2, b2c]
    kwargs = {}
    body = _mlp_body
    if out_prev is not None:
        in_specs = in_specs + [pl.BlockSpec(memory_space=pl.ANY)]
        args = args + [out_prev]
        kwargs["input_output_aliases"] = {7: 0}
        body = _mlp_body_alias
    return pl.pallas_call(
        body,
        grid=(grid,),
        in_specs=in_specs,
        out_specs=pl.BlockSpec((_A_BLK, D_SIDE, _HB),
                               lambda i, h=half: (i, 0, h)),
        out_shape=jax.ShapeDtypeStruct((AA_H, D_SIDE, B), jnp.float32),
        scratch_shapes=[pltpu.VMEM((D_SIDE, _HB), jnp.float32)],
        **kwargs,
    )(*args)


def kernel(side, table, W1, b1, gamma, beta, W2, b2):
    idx = side.astype(jnp.int32)
    tableT = table.T
    b1c = b1.reshape(D_SIDE, 1)
    gammac = gamma.reshape(D_SIDE, 1)
    betac = beta.reshape(D_SIDE, 1)
    b2c = b2.reshape(D_SIDE, 1)
    gather = _make_sc_gather(_HB)
    emb0 = gather(idx[:_HB], tableT)
    emb1 = gather(idx[_HB:], tableT)
    out0 = _tc_half(emb0, W1, b1c, gammac, betac, W2, b2c, half=0)
    out = _tc_half(emb1, W1, b1c, gammac, betac, W2, b2c, half=1,
                   out_prev=out0)
    return out.transpose(2, 0, 1)
